# baseline (speedup 1.0000x reference)
import jax
import jax.numpy as jnp
from jax import lax
from jax.experimental import pallas as pl
from jax.experimental.pallas import tpu as pltpu
from jax.experimental.pallas import tpu_sc as plsc

_PLACEHOLDER_TOKEN = 500
_L = 16
_CHUNK = 256
_NBUF = 16


def _copy_body(emb_hbm, out_hbm, bufs, in_sems, out_sems):
    rows = emb_hbm.shape[0]
    nchunk = rows // _CHUNK

    def in_dma(t, s):
        return pltpu.make_async_copy(
            emb_hbm.at[pl.ds(t * _CHUNK, _CHUNK)], bufs.at[s], in_sems.at[s])

    def out_dma(t, s):
        return pltpu.make_async_copy(
            bufs.at[s], out_hbm.at[pl.ds(t * _CHUNK, _CHUNK)], out_sems.at[s])

    for s in range(_NBUF):
        in_dma(s, s).start()

    def step(t, carry):
        slot = lax.rem(t, _NBUF)
        in_dma(t, slot).wait()

        @pl.when(t >= _NBUF)
        def _():
            out_dma(t - _NBUF, slot).wait()

        out_dma(t, slot).start()

        @pl.when(t + _NBUF < nchunk)
        def _():
            in_dma(t + _NBUF, slot).start()

        return carry

    lax.fori_loop(0, nchunk, step, 0)
    for s in range(_NBUF):
        t = nchunk - _NBUF + s
        out_dma(t, t % _NBUF).wait()


def _tc_copy(emb):
    rows, d = emb.shape
    return pl.pallas_call(
        _copy_body,
        in_specs=[pl.BlockSpec(memory_space=pl.ANY)],
        out_specs=pl.BlockSpec(memory_space=pl.ANY),
        out_shape=jax.ShapeDtypeStruct((rows, d), emb.dtype),
        scratch_shapes=[
            pltpu.VMEM((_NBUF, _CHUNK, d), jnp.float32),
            pltpu.SemaphoreType.DMA((_NBUF,)),
            pltpu.SemaphoreType.DMA((_NBUF,)),
        ],
    )(emb)


def _scatter_body(tok_hbm, vec_hbm, out_ref, tok_v, vec_v, sem):
    rows, d = out_ref.shape
    nc = 2
    ns = 16
    wid = lax.axis_index("s") * nc + lax.axis_index("c")
    rows_per_tile = rows // (nc * ns)
    base = wid * rows_per_tile

    pltpu.sync_copy(tok_hbm.at[pl.ds(base, rows_per_tile)], tok_v)
    pltpu.sync_copy(vec_hbm.at[0], vec_v)

    lanes = lax.iota(jnp.int32, _L)

    for v in range(rows_per_tile // _L):
        tok16 = tok_v[pl.ds(v * _L, _L)]
        match = tok16 == _PLACEHOLDER_TOKEN
        m = jnp.where(match, 1, 0)
        any_match = plsc.all_reduce_population_count(match)[0]

        @pl.when(any_match > 0)
        def _():
            def cond(mm):
                return plsc.all_reduce_population_count(mm > 0)[0] > 0

            def body(mm):
                lane_v = plsc.all_reduce_ffs(mm > 0)
                row = base + v * _L + lane_v[0]
                pltpu.make_async_copy(vec_v, out_ref.at[row], sem).start()
                pltpu.make_async_copy(vec_v, out_ref.at[row], sem).wait()
                return jnp.where(lanes == lane_v, 0, mm)

            lax.while_loop(cond, body, m)


def _sc_scatter(tok, vec, out_ref):
    rows, d = out_ref.shape
    mesh = plsc.VectorSubcoreMesh(core_axis_name="c", subcore_axis_name="s")
    run = pl.kernel(
        _scatter_body,
        out_type=(),
        mesh=mesh,
        scratch_types=[
            pltpu.VMEM((rows // 32,), jnp.int32),
            pltpu.VMEM((d,), jnp.float32),
            pltpu.SemaphoreType.DMA,
        ],
        compiler_params=pltpu.CompilerParams(needs_layout_passes=False),
    )
    run(tok, vec, out_ref)


def kernel(tokenized_text, embedded_text, placeholder_embedding):
    b, n, d = embedded_text.shape
    rows = b * n
    emb = embedded_text.reshape(rows, d)
    tok = tokenized_text.reshape(rows)
    copied = _tc_copy(emb)
    out_ref = jax.new_ref(copied)
    _sc_scatter(tok, placeholder_embedding, out_ref)
    return out_ref[...].reshape(b, n, d)
